# baseline (speedup 1.0000x reference)
import jax
import jax.numpy as jnp
import numpy as np
from jax.experimental import pallas as pl
from jax.experimental.pallas import tpu as pltpu

N_EMB = 8192
EMB_DIM = 32
BETA = 0.25
THRESHOLD = 1.0
ROWS = 16384
T = 256
NT = ROWS // T

_PERM = np.asarray(jax.random.permutation(jax.random.key(42), ROWS))[:N_EMB]


def _k1(x_ref, embT_ref, idx_ref, seg_ref, stats_ref):
    i = pl.program_id(0)
    rows = x_ref[...]
    embT = embT_ref[...]
    sim = jnp.dot(rows, embT, preferred_element_type=jnp.float32)
    s1 = jnp.sum(rows * rows, axis=1, keepdims=True)
    s2 = jnp.sum(embT * embT, axis=0, keepdims=True)
    dist = s1 + s2 - 2.0 * sim
    mind = jnp.min(dist, axis=1, keepdims=True)
    codes = jax.lax.broadcasted_iota(jnp.int32, (T, N_EMB), 1)
    idx = jnp.min(jnp.where(dist == mind, codes, jnp.int32(N_EMB)), axis=1)
    idx_ref[0, 0, :] = idx
    one_hot = (codes == idx[:, None]).astype(jnp.float32)
    rows_aug = jnp.concatenate(
        [rows, jnp.ones((T, 1), jnp.float32)], axis=1)
    seg = jax.lax.dot_general(
        one_hot, rows_aug, (((0,), (0,)), ((), ())),
        preferred_element_type=jnp.float32)

    @pl.when(i == 0)
    def _():
        seg_ref[...] = jnp.zeros_like(seg_ref)
        stats_ref[0] = 0.0
        stats_ref[1] = 0.0
        stats_ref[2] = 0.0

    seg_ref[...] += seg
    stats_ref[0] += jnp.sum(rows)
    stats_ref[1] += jnp.sum(s1)
    stats_ref[2] += jnp.sum(jnp.nan_to_num(mind))


def _k3(seg_ref, emb_ref, krand_ref, newk_ref, sc_ref, used_ref):
    seg = seg_ref[...]
    ksum_new = seg[:, :EMB_DIM]
    kelem_new = seg[:, EMB_DIM:EMB_DIM + 1]
    emb = emb_ref[...]
    k_sum = BETA * emb + (1.0 - BETA) * ksum_new
    k_elem = BETA * 1.0 + (1.0 - BETA) * kelem_new
    usage = (k_elem >= THRESHOLD).astype(jnp.float32)
    new_k = usage * (k_sum / k_elem) + (1.0 - usage) * krand_ref[...]
    newk_ref[...] = new_k
    prob = kelem_new / jnp.sum(kelem_new)
    sc_ref[0] = -jnp.sum(prob * jnp.log(prob + 1e-8))
    sc_ref[1] = jnp.sum(usage)
    diff = new_k - emb
    sc_ref[2] = jnp.sum(diff * diff)
    used_ref[0] = jnp.sum((kelem_new >= THRESHOLD).astype(jnp.int32))


def _k2(idx_ref, x_ref, newk_ref, q_ref, comm_ref):
    i = pl.program_id(0)
    idx = idx_ref[0, 0, :]
    rows = x_ref[...]
    newk = newk_ref[...]
    codes = jax.lax.broadcasted_iota(jnp.int32, (T, N_EMB), 1)
    one_hot = (codes == idx[:, None]).astype(jnp.float32)
    q = jnp.dot(one_hot, newk, preferred_element_type=jnp.float32)
    q_ref[...] = q

    @pl.when(i == 0)
    def _():
        comm_ref[0] = 0.0

    d = q - rows
    comm_ref[0] += jnp.sum(d * d)


def kernel(x, embeddings):
    xt = jnp.swapaxes(x, 1, -1)
    flat_x = xt.reshape(ROWS, EMB_DIM)
    embT = embeddings.T

    idx3, seg, stats = pl.pallas_call(
        _k1,
        grid=(NT,),
        in_specs=[
            pl.BlockSpec((T, EMB_DIM), lambda i: (i, 0)),
            pl.BlockSpec((EMB_DIM, N_EMB), lambda i: (0, 0)),
        ],
        out_specs=[
            pl.BlockSpec((1, 1, T), lambda i: (i, 0, 0)),
            pl.BlockSpec((N_EMB, EMB_DIM + 1), lambda i: (0, 0)),
            pl.BlockSpec(memory_space=pltpu.SMEM),
        ],
        out_shape=[
            jax.ShapeDtypeStruct((NT, 1, T), jnp.int32),
            jax.ShapeDtypeStruct((N_EMB, EMB_DIM + 1), jnp.float32),
            jax.ShapeDtypeStruct((4,), jnp.float32),
        ],
    )(flat_x, embT)

    k_rand = jnp.take(flat_x, _PERM, axis=0)

    new_k, sc, usedc = pl.pallas_call(
        _k3,
        in_specs=[
            pl.BlockSpec((N_EMB, EMB_DIM + 1), lambda: (0, 0)),
            pl.BlockSpec((N_EMB, EMB_DIM), lambda: (0, 0)),
            pl.BlockSpec((N_EMB, EMB_DIM), lambda: (0, 0)),
        ],
        out_specs=[
            pl.BlockSpec((N_EMB, EMB_DIM), lambda: (0, 0)),
            pl.BlockSpec(memory_space=pltpu.SMEM),
            pl.BlockSpec(memory_space=pltpu.SMEM),
        ],
        out_shape=[
            jax.ShapeDtypeStruct((N_EMB, EMB_DIM), jnp.float32),
            jax.ShapeDtypeStruct((4,), jnp.float32),
            jax.ShapeDtypeStruct((1,), jnp.int32),
        ],
    )(seg, embeddings, k_rand)

    q_flat, comm = pl.pallas_call(
        _k2,
        grid=(NT,),
        in_specs=[
            pl.BlockSpec((1, 1, T), lambda i: (i, 0, 0)),
            pl.BlockSpec((T, EMB_DIM), lambda i: (i, 0)),
            pl.BlockSpec((N_EMB, EMB_DIM), lambda i: (0, 0)),
        ],
        out_specs=[
            pl.BlockSpec((T, EMB_DIM), lambda i: (i, 0)),
            pl.BlockSpec(memory_space=pltpu.SMEM),
        ],
        out_shape=[
            jax.ShapeDtypeStruct((ROWS, EMB_DIM), jnp.float32),
            jax.ShapeDtypeStruct((1,), jnp.float32),
        ],
    )(idx3, flat_x, new_k)

    quantized = jnp.swapaxes(q_flat.reshape(xt.shape), 1, -1)
    out = x + jax.lax.stop_gradient(quantized - x)

    n = float(ROWS * EMB_DIM)
    mean = stats[0] / n
    prenorm = jnp.sqrt(jnp.maximum(stats[1] - n * mean * mean, 0.0) / n)
    fit = stats[2] / float(ROWS)
    loss = BETA * comm[0] / n
    entropy = sc[0]
    usage_sum = sc[1]
    dk = jnp.nan_to_num(jnp.sqrt(sc[2]) / np.sqrt(float(N_EMB * EMB_DIM)))
    used_curr = usedc[0]
    return (out, quantized, loss, fit, prenorm, entropy, used_curr,
            usage_sum, dk)
